# baseline (speedup 1.0000x reference)
import functools

import jax
import jax.numpy as jnp
from jax.experimental import pallas as pl
from jax.experimental.pallas import tpu as pltpu
from jax.experimental.pallas import tpu_sc as plsc

_ROWS = 2048
_GW = 128


def _table_kernel(tok_ref, pos_ref, w_ref, b_ref, lse_ref, posb_ref):
    t = pl.program_id(0)
    posb = jax.lax.dot_general(
        pos_ref[pl.ds(t, 1), :], w_ref[...], (((1,), (0,)), ((), ())),
        preferred_element_type=jnp.float32,
        precision=jax.lax.Precision.DEFAULT) + b_ref[...]
    slab = jax.lax.dot_general(
        tok_ref[...], w_ref[...], (((1,), (0,)), ((), ())),
        preferred_element_type=jnp.float32,
        precision=jax.lax.Precision.DEFAULT) + posb
    m = jnp.max(slab, axis=1, keepdims=True)
    lse = jnp.log(jnp.sum(jnp.exp(slab - m), axis=1, keepdims=True)) + m
    lse_ref[...] = lse[None]
    posb_ref[...] = posb[None]


def _main_kernel(feat_ref, tgt_ref, a2_ref, pos_ref, mask_ref,
                 w_ref, logits_ref, part_ref):
    r, v = logits_ref.shape
    c = pos_ref.shape[1]

    feat = feat_ref[...]
    x = feat[:, 0:c] + pos_ref[...]

    tgt = tgt_ref[...]
    vocab_iota = jax.lax.broadcasted_iota(jnp.int32, (r, v), 1)
    oh2 = (tgt == vocab_iota).astype(jnp.bfloat16)
    g2 = jax.lax.dot_general(
        oh2, a2_ref[...], (((1,), (0,)), ((), ())),
        preferred_element_type=jnp.float32,
        precision=jax.lax.Precision.DEFAULT)

    logits_ref[...] = jax.lax.dot_general(
        x, w_ref[...], (((1,), (0,)), ((), ())),
        preferred_element_type=jnp.float32,
        precision=jax.lax.Precision.DEFAULT)

    part = (jnp.sum((feat[:, c:c + 8] - g2[:, c:c + 8]) * mask_ref[...])
            - jnp.sum(x * g2[:, 0:c]))
    part_ref[...] = jnp.full((1, 1, 128), part, jnp.float32)


def _loss_reduce_kernel(part_ref, loss_ref, *, n):
    total = jnp.sum(part_ref[...][:, :, 0])
    loss_ref[...] = jnp.full((1, 1), total / n, jnp.float32)


def kernel(idx, targets, tok_table, pos_table, W, b):
    B, T = idx.shape
    V, C = tok_table.shape
    n = B * T
    r = _ROWS
    nblocks = n // r

    tgt_r = targets.reshape(n, 1).astype(jnp.int32)
    idx_flat = idx.reshape(1, n).astype(jnp.int32)
    b2 = b.reshape(1, V)

    lse_tab, posb = pl.pallas_call(
        _table_kernel,
        grid=(T,),
        in_specs=[
            pl.BlockSpec((V, C), lambda t: (0, 0)),
            pl.BlockSpec((T, C), lambda t: (0, 0)),
            pl.BlockSpec((C, V), lambda t: (0, 0)),
            pl.BlockSpec((1, V), lambda t: (0, 0)),
        ],
        out_specs=[
            pl.BlockSpec((1, V, 1), lambda t: (t, 0, 0)),
            pl.BlockSpec((1, 1, V), lambda t: (t, 0, 0)),
        ],
        out_shape=[
            jax.ShapeDtypeStruct((T, V, 1), jnp.float32),
            jax.ShapeDtypeStruct((T, 1, V), jnp.float32),
        ],
    )(tok_table, pos_table, W, b2)

    ones_col = jnp.ones((V, 1), jnp.float32)
    feat_tab = jnp.concatenate(
        [tok_table, ones_col, lse_tab.reshape(T, V).T,
         jnp.zeros((V, 128 - C - 1 - T), jnp.float32)],
        axis=1)
    posw_t = posb.reshape(T, V) - b2
    a2 = jnp.concatenate(
        [W.T, b.reshape(V, 1), posw_t.T], axis=1).astype(jnp.bfloat16)
    w_aug = jnp.concatenate([W, b2], axis=0)
    pos_aug = jnp.concatenate(
        [pos_table, jnp.zeros((T, 1), jnp.float32)], axis=1)
    pos_tile = jnp.tile(pos_aug, (r // T, 1))
    mask_tile = jnp.tile(jnp.eye(T, dtype=jnp.float32), (r // T, 1))

    sc_mesh = plsc.VectorSubcoreMesh(core_axis_name="core",
                                     subcore_axis_name="subcore")

    @pl.kernel(out_type=jax.ShapeDtypeStruct((n, 128), jnp.float32),
               mesh=sc_mesh)
    def _sc_gather_feat(feat_hbm, i_hbm, o_hbm):
        def body(i_vmem, o_vmem):
            pltpu.sync_copy(feat_hbm.at[i_vmem.at[0]], o_vmem)

        pltpu.emit_pipeline(
            body,
            grid=(n // _GW,),
            in_specs=[pl.BlockSpec((1, _GW), lambda i: (0, i))],
            out_specs=[pl.BlockSpec((_GW, 128), lambda i: (i, 0))],
            core_axis_name=("core", "subcore"),
            dimension_semantics=(pltpu.PARALLEL,),
        )(i_hbm, o_hbm)

    feat = _sc_gather_feat(feat_tab, idx_flat)

    c1 = C + 1
    logits, parts = pl.pallas_call(
        _main_kernel,
        grid=(nblocks,),
        in_specs=[
            pl.BlockSpec((r, 128), lambda i: (i, 0)),
            pl.BlockSpec((r, 1), lambda i: (i, 0)),
            pl.BlockSpec((V, c1 + T), lambda i: (0, 0)),
            pl.BlockSpec((r, c1), lambda i: (0, 0)),
            pl.BlockSpec((r, T), lambda i: (0, 0)),
            pl.BlockSpec((c1, V), lambda i: (0, 0)),
        ],
        out_specs=[
            pl.BlockSpec((r, V), lambda i: (i, 0)),
            pl.BlockSpec((1, 1, 128), lambda i: (i, 0, 0)),
        ],
        out_shape=[
            jax.ShapeDtypeStruct((n, V), jnp.float32),
            jax.ShapeDtypeStruct((nblocks, 1, 128), jnp.float32),
        ],
        compiler_params=pltpu.CompilerParams(
            dimension_semantics=("parallel",)),
    )(feat, tgt_r, a2, pos_tile, mask_tile, w_aug)

    loss = pl.pallas_call(
        functools.partial(_loss_reduce_kernel, n=n),
        out_shape=jax.ShapeDtypeStruct((1, 1), jnp.float32),
    )(parts)
    return logits, loss[0, 0]


# speedup vs baseline: 1.0031x; 1.0031x over previous
import functools

import jax
import jax.numpy as jnp
from jax.experimental import pallas as pl
from jax.experimental.pallas import tpu as pltpu
from jax.experimental.pallas import tpu_sc as plsc

_ROWS = 2048
_GW = 256


def _table_kernel(tok_ref, pos_ref, w_ref, b_ref, lse_ref, posb_ref):
    t = pl.program_id(0)
    posb = jax.lax.dot_general(
        pos_ref[pl.ds(t, 1), :], w_ref[...], (((1,), (0,)), ((), ())),
        preferred_element_type=jnp.float32,
        precision=jax.lax.Precision.DEFAULT) + b_ref[...]
    slab = jax.lax.dot_general(
        tok_ref[...], w_ref[...], (((1,), (0,)), ((), ())),
        preferred_element_type=jnp.float32,
        precision=jax.lax.Precision.DEFAULT) + posb
    m = jnp.max(slab, axis=1, keepdims=True)
    lse = jnp.log(jnp.sum(jnp.exp(slab - m), axis=1, keepdims=True)) + m
    lse_ref[...] = lse[None]
    posb_ref[...] = posb[None]


def _main_kernel(feat_ref, tgt_ref, a2_ref, pos_ref, mask_ref,
                 w_ref, logits_ref, part_ref):
    r, v = logits_ref.shape
    c = pos_ref.shape[1]

    feat = feat_ref[...]
    x = feat[:, 0:c] + pos_ref[...]

    tgt = tgt_ref[...]
    vocab_iota = jax.lax.broadcasted_iota(jnp.int32, (r, v), 1)
    oh2 = (tgt == vocab_iota).astype(jnp.bfloat16)
    g2 = jax.lax.dot_general(
        oh2, a2_ref[...], (((1,), (0,)), ((), ())),
        preferred_element_type=jnp.float32,
        precision=jax.lax.Precision.DEFAULT)

    logits_ref[...] = jax.lax.dot_general(
        x, w_ref[...], (((1,), (0,)), ((), ())),
        preferred_element_type=jnp.float32,
        precision=jax.lax.Precision.DEFAULT)

    part = (jnp.sum((feat[:, c:c + 8] - g2[:, c:c + 8]) * mask_ref[...])
            - jnp.sum(x * g2[:, 0:c]))
    part_ref[...] = jnp.full((1, 1, 128), part, jnp.float32)


def _loss_reduce_kernel(part_ref, loss_ref, *, n):
    total = jnp.sum(part_ref[...][:, :, 0])
    loss_ref[...] = jnp.full((1, 1), total / n, jnp.float32)


def kernel(idx, targets, tok_table, pos_table, W, b):
    B, T = idx.shape
    V, C = tok_table.shape
    n = B * T
    r = _ROWS
    nblocks = n // r

    tgt_r = targets.reshape(n, 1).astype(jnp.int32)
    idx_flat = idx.reshape(1, n).astype(jnp.int32)
    b2 = b.reshape(1, V)

    lse_tab, posb = pl.pallas_call(
        _table_kernel,
        grid=(T,),
        in_specs=[
            pl.BlockSpec((V, C), lambda t: (0, 0)),
            pl.BlockSpec((T, C), lambda t: (0, 0)),
            pl.BlockSpec((C, V), lambda t: (0, 0)),
            pl.BlockSpec((1, V), lambda t: (0, 0)),
        ],
        out_specs=[
            pl.BlockSpec((1, V, 1), lambda t: (t, 0, 0)),
            pl.BlockSpec((1, 1, V), lambda t: (t, 0, 0)),
        ],
        out_shape=[
            jax.ShapeDtypeStruct((T, V, 1), jnp.float32),
            jax.ShapeDtypeStruct((T, 1, V), jnp.float32),
        ],
    )(tok_table, pos_table, W, b2)

    ones_col = jnp.ones((V, 1), jnp.float32)
    feat_tab = jnp.concatenate(
        [tok_table, ones_col, lse_tab.reshape(T, V).T,
         jnp.zeros((V, 128 - C - 1 - T), jnp.float32)],
        axis=1)
    posw_t = posb.reshape(T, V) - b2
    a2 = jnp.concatenate(
        [W.T, b.reshape(V, 1), posw_t.T], axis=1).astype(jnp.bfloat16)
    w_aug = jnp.concatenate([W, b2], axis=0)
    pos_aug = jnp.concatenate(
        [pos_table, jnp.zeros((T, 1), jnp.float32)], axis=1)
    pos_tile = jnp.tile(pos_aug, (r // T, 1))
    mask_tile = jnp.tile(jnp.eye(T, dtype=jnp.float32), (r // T, 1))

    sc_mesh = plsc.VectorSubcoreMesh(core_axis_name="core",
                                     subcore_axis_name="subcore")

    @pl.kernel(out_type=jax.ShapeDtypeStruct((n, 128), jnp.float32),
               mesh=sc_mesh)
    def _sc_gather_feat(feat_hbm, i_hbm, o_hbm):
        def body(i_vmem, o_vmem):
            pltpu.sync_copy(feat_hbm.at[i_vmem.at[0]], o_vmem)

        pltpu.emit_pipeline(
            body,
            grid=(n // _GW,),
            in_specs=[pl.BlockSpec((1, _GW), lambda i: (0, i))],
            out_specs=[pl.BlockSpec((_GW, 128), lambda i: (i, 0))],
            core_axis_name=("core", "subcore"),
            dimension_semantics=(pltpu.PARALLEL,),
        )(i_hbm, o_hbm)

    feat = _sc_gather_feat(feat_tab, idx_flat)

    c1 = C + 1
    logits, parts = pl.pallas_call(
        _main_kernel,
        grid=(nblocks,),
        in_specs=[
            pl.BlockSpec((r, 128), lambda i: (i, 0)),
            pl.BlockSpec((r, 1), lambda i: (i, 0)),
            pl.BlockSpec((V, c1 + T), lambda i: (0, 0)),
            pl.BlockSpec((r, c1), lambda i: (0, 0)),
            pl.BlockSpec((r, T), lambda i: (0, 0)),
            pl.BlockSpec((c1, V), lambda i: (0, 0)),
        ],
        out_specs=[
            pl.BlockSpec((r, V), lambda i: (i, 0)),
            pl.BlockSpec((1, 1, 128), lambda i: (i, 0, 0)),
        ],
        out_shape=[
            jax.ShapeDtypeStruct((n, V), jnp.float32),
            jax.ShapeDtypeStruct((nblocks, 1, 128), jnp.float32),
        ],
        compiler_params=pltpu.CompilerParams(
            dimension_semantics=("parallel",)),
    )(feat, tgt_r, a2, pos_tile, mask_tile, w_aug)

    loss = pl.pallas_call(
        functools.partial(_loss_reduce_kernel, n=n),
        out_shape=jax.ShapeDtypeStruct((1, 1), jnp.float32),
    )(parts)
    return logits, loss[0, 0]


# speedup vs baseline: 1.0104x; 1.0073x over previous
import functools

import jax
import jax.numpy as jnp
from jax.experimental import pallas as pl
from jax.experimental.pallas import tpu as pltpu
from jax.experimental.pallas import tpu_sc as plsc

_ROWS = 4096
_GW = 256


def _table_kernel(tok_ref, pos_ref, w_ref, b_ref, lse_ref, posb_ref):
    t = pl.program_id(0)
    posb = jax.lax.dot_general(
        pos_ref[pl.ds(t, 1), :], w_ref[...], (((1,), (0,)), ((), ())),
        preferred_element_type=jnp.float32,
        precision=jax.lax.Precision.DEFAULT) + b_ref[...]
    slab = jax.lax.dot_general(
        tok_ref[...], w_ref[...], (((1,), (0,)), ((), ())),
        preferred_element_type=jnp.float32,
        precision=jax.lax.Precision.DEFAULT) + posb
    m = jnp.max(slab, axis=1, keepdims=True)
    lse = jnp.log(jnp.sum(jnp.exp(slab - m), axis=1, keepdims=True)) + m
    lse_ref[...] = lse[None]
    posb_ref[...] = posb[None]


def _main_kernel(feat_ref, tgt_ref, a2_ref, pos_ref, mask_ref,
                 w_ref, logits_ref, part_ref):
    r, v = logits_ref.shape
    c = pos_ref.shape[1]

    feat = feat_ref[...]
    x = feat[:, 0:c] + pos_ref[...]

    tgt = tgt_ref[...]
    vocab_iota = jax.lax.broadcasted_iota(jnp.int32, (r, v), 1)
    oh2 = (tgt == vocab_iota).astype(jnp.bfloat16)
    g2 = jax.lax.dot_general(
        oh2, a2_ref[...], (((1,), (0,)), ((), ())),
        preferred_element_type=jnp.float32,
        precision=jax.lax.Precision.DEFAULT)

    logits_ref[...] = jax.lax.dot_general(
        x, w_ref[...], (((1,), (0,)), ((), ())),
        preferred_element_type=jnp.float32,
        precision=jax.lax.Precision.DEFAULT)

    part = (jnp.sum((feat[:, c:c + 8] - g2[:, c:c + 8]) * mask_ref[...])
            - jnp.sum(x * g2[:, 0:c]))
    part_ref[...] = jnp.full((1, 1, 128), part, jnp.float32)


def _loss_reduce_kernel(part_ref, loss_ref, *, n):
    total = jnp.sum(part_ref[...][:, :, 0])
    loss_ref[...] = jnp.full((1, 1), total / n, jnp.float32)


def kernel(idx, targets, tok_table, pos_table, W, b):
    B, T = idx.shape
    V, C = tok_table.shape
    n = B * T
    r = _ROWS
    nblocks = n // r

    tgt_r = targets.reshape(n, 1).astype(jnp.int32)
    idx_flat = idx.reshape(1, n).astype(jnp.int32)
    b2 = b.reshape(1, V)

    lse_tab, posb = pl.pallas_call(
        _table_kernel,
        grid=(T,),
        in_specs=[
            pl.BlockSpec((V, C), lambda t: (0, 0)),
            pl.BlockSpec((T, C), lambda t: (0, 0)),
            pl.BlockSpec((C, V), lambda t: (0, 0)),
            pl.BlockSpec((1, V), lambda t: (0, 0)),
        ],
        out_specs=[
            pl.BlockSpec((1, V, 1), lambda t: (t, 0, 0)),
            pl.BlockSpec((1, 1, V), lambda t: (t, 0, 0)),
        ],
        out_shape=[
            jax.ShapeDtypeStruct((T, V, 1), jnp.float32),
            jax.ShapeDtypeStruct((T, 1, V), jnp.float32),
        ],
    )(tok_table, pos_table, W, b2)

    ones_col = jnp.ones((V, 1), jnp.float32)
    feat_tab = jnp.concatenate(
        [tok_table, ones_col, lse_tab.reshape(T, V).T,
         jnp.zeros((V, 128 - C - 1 - T), jnp.float32)],
        axis=1)
    posw_t = posb.reshape(T, V) - b2
    a2 = jnp.concatenate(
        [W.T, b.reshape(V, 1), posw_t.T], axis=1).astype(jnp.bfloat16)
    w_aug = jnp.concatenate([W, b2], axis=0)
    pos_aug = jnp.concatenate(
        [pos_table, jnp.zeros((T, 1), jnp.float32)], axis=1)
    pos_tile = jnp.tile(pos_aug, (r // T, 1))
    mask_tile = jnp.tile(jnp.eye(T, dtype=jnp.float32), (r // T, 1))

    sc_mesh = plsc.VectorSubcoreMesh(core_axis_name="core",
                                     subcore_axis_name="subcore")

    @pl.kernel(out_type=jax.ShapeDtypeStruct((n, 128), jnp.float32),
               mesh=sc_mesh)
    def _sc_gather_feat(feat_hbm, i_hbm, o_hbm):
        def body(i_vmem, o_vmem):
            pltpu.sync_copy(feat_hbm.at[i_vmem.at[0]], o_vmem)

        pltpu.emit_pipeline(
            body,
            grid=(n // _GW,),
            in_specs=[pl.BlockSpec((1, _GW), lambda i: (0, i))],
            out_specs=[pl.BlockSpec((_GW, 128), lambda i: (i, 0))],
            core_axis_name=("core", "subcore"),
            dimension_semantics=(pltpu.PARALLEL,),
        )(i_hbm, o_hbm)

    feat = _sc_gather_feat(feat_tab, idx_flat)

    c1 = C + 1
    logits, parts = pl.pallas_call(
        _main_kernel,
        grid=(nblocks,),
        in_specs=[
            pl.BlockSpec((r, 128), lambda i: (i, 0)),
            pl.BlockSpec((r, 1), lambda i: (i, 0)),
            pl.BlockSpec((V, c1 + T), lambda i: (0, 0)),
            pl.BlockSpec((r, c1), lambda i: (0, 0)),
            pl.BlockSpec((r, T), lambda i: (0, 0)),
            pl.BlockSpec((c1, V), lambda i: (0, 0)),
        ],
        out_specs=[
            pl.BlockSpec((r, V), lambda i: (i, 0)),
            pl.BlockSpec((1, 1, 128), lambda i: (i, 0, 0)),
        ],
        out_shape=[
            jax.ShapeDtypeStruct((n, V), jnp.float32),
            jax.ShapeDtypeStruct((nblocks, 1, 128), jnp.float32),
        ],
        compiler_params=pltpu.CompilerParams(
            dimension_semantics=("parallel",)),
    )(feat, tgt_r, a2, pos_tile, mask_tile, w_aug)

    loss = pl.pallas_call(
        functools.partial(_loss_reduce_kernel, n=n),
        out_shape=jax.ShapeDtypeStruct((1, 1), jnp.float32),
    )(parts)
    return logits, loss[0, 0]


# speedup vs baseline: 1.0118x; 1.0014x over previous
import functools

import jax
import jax.numpy as jnp
from jax.experimental import pallas as pl
from jax.experimental.pallas import tpu as pltpu

_ROWS = 2048


def _table_kernel(tok_ref, pos_ref, w_ref, b_ref, lse_ref, posb_ref):
    t = pl.program_id(0)
    posb = jax.lax.dot_general(
        pos_ref[pl.ds(t, 1), :], w_ref[...], (((1,), (0,)), ((), ())),
        preferred_element_type=jnp.float32,
        precision=jax.lax.Precision.DEFAULT) + b_ref[...]
    slab = jax.lax.dot_general(
        tok_ref[...], w_ref[...], (((1,), (0,)), ((), ())),
        preferred_element_type=jnp.float32,
        precision=jax.lax.Precision.DEFAULT) + posb
    m = jnp.max(slab, axis=1, keepdims=True)
    lse = jnp.log(jnp.sum(jnp.exp(slab - m), axis=1, keepdims=True)) + m
    lse_ref[...] = lse[None]
    posb_ref[...] = posb[None]


def _main_kernel(idx_ref, tgt_ref, a1_ref, a2_ref, pos_ref, mask_ref,
                 w_ref, logits_ref, part_ref):
    r, v = logits_ref.shape
    c = pos_ref.shape[1]

    ids = idx_ref[...]
    tgt = tgt_ref[...]
    vocab_iota = jax.lax.broadcasted_iota(jnp.int32, (r, v), 1)

    oh1 = (ids == vocab_iota).astype(jnp.bfloat16)
    g1 = jax.lax.dot_general(
        oh1, a1_ref[...], (((1,), (0,)), ((), ())),
        preferred_element_type=jnp.float32,
        precision=jax.lax.Precision.DEFAULT)
    x = g1[:, 0:c] + pos_ref[...]

    oh2 = (tgt == vocab_iota).astype(jnp.bfloat16)
    g2 = jax.lax.dot_general(
        oh2, a2_ref[...], (((1,), (0,)), ((), ())),
        preferred_element_type=jnp.float32,
        precision=jax.lax.Precision.DEFAULT)

    logits_ref[...] = jax.lax.dot_general(
        x, w_ref[...], (((1,), (0,)), ((), ())),
        preferred_element_type=jnp.float32,
        precision=jax.lax.Precision.DEFAULT)

    part = (jnp.sum((g1[:, c:c + 8] - g2[:, c:c + 8]) * mask_ref[...])
            - jnp.sum(x * g2[:, 0:c]))
    part_ref[...] = jnp.full((1, 1, 128), part, jnp.float32)


def _loss_reduce_kernel(part_ref, loss_ref, *, n):
    total = jnp.sum(part_ref[...][:, :, 0])
    loss_ref[...] = jnp.full((1, 1), total / n, jnp.float32)


def kernel(idx, targets, tok_table, pos_table, W, b):
    B, T = idx.shape
    V, C = tok_table.shape
    n = B * T
    r = _ROWS
    nblocks = n // r

    idx_r = idx.reshape(n, 1).astype(jnp.int32)
    tgt_r = targets.reshape(n, 1).astype(jnp.int32)
    b2 = b.reshape(1, V)

    lse_tab, posb = pl.pallas_call(
        _table_kernel,
        grid=(T,),
        in_specs=[
            pl.BlockSpec((V, C), lambda t: (0, 0)),
            pl.BlockSpec((T, C), lambda t: (0, 0)),
            pl.BlockSpec((C, V), lambda t: (0, 0)),
            pl.BlockSpec((1, V), lambda t: (0, 0)),
        ],
        out_specs=[
            pl.BlockSpec((1, V, 1), lambda t: (t, 0, 0)),
            pl.BlockSpec((1, 1, V), lambda t: (t, 0, 0)),
        ],
        out_shape=[
            jax.ShapeDtypeStruct((T, V, 1), jnp.float32),
            jax.ShapeDtypeStruct((T, 1, V), jnp.float32),
        ],
    )(tok_table, pos_table, W, b2)

    ones_col = jnp.ones((V, 1), jnp.float32)
    a1 = jnp.concatenate(
        [tok_table, ones_col, lse_tab.reshape(T, V).T],
        axis=1).astype(jnp.bfloat16)
    posw_t = posb.reshape(T, V) - b2
    a2 = jnp.concatenate(
        [W.T, b.reshape(V, 1), posw_t.T], axis=1).astype(jnp.bfloat16)
    w_aug = jnp.concatenate([W, b2], axis=0)
    pos_aug = jnp.concatenate(
        [pos_table, jnp.zeros((T, 1), jnp.float32)], axis=1)
    pos_tile = jnp.tile(pos_aug, (r // T, 1))
    mask_tile = jnp.tile(jnp.eye(T, dtype=jnp.float32), (r // T, 1))

    c1 = C + 1
    logits, parts = pl.pallas_call(
        _main_kernel,
        grid=(nblocks,),
        in_specs=[
            pl.BlockSpec((r, 1), lambda i: (i, 0)),
            pl.BlockSpec((r, 1), lambda i: (i, 0)),
            pl.BlockSpec((V, c1 + T), lambda i: (0, 0)),
            pl.BlockSpec((V, c1 + T), lambda i: (0, 0)),
            pl.BlockSpec((r, c1), lambda i: (0, 0)),
            pl.BlockSpec((r, T), lambda i: (0, 0)),
            pl.BlockSpec((c1, V), lambda i: (0, 0)),
        ],
        out_specs=[
            pl.BlockSpec((r, V), lambda i: (i, 0)),
            pl.BlockSpec((1, 1, 128), lambda i: (i, 0, 0)),
        ],
        out_shape=[
            jax.ShapeDtypeStruct((n, V), jnp.float32),
            jax.ShapeDtypeStruct((nblocks, 1, 128), jnp.float32),
        ],
        compiler_params=pltpu.CompilerParams(
            dimension_semantics=("parallel",)),
    )(idx_r, tgt_r, a1, a2, pos_tile, mask_tile, w_aug)

    loss = pl.pallas_call(
        functools.partial(_loss_reduce_kernel, n=n),
        out_shape=jax.ShapeDtypeStruct((1, 1), jnp.float32),
    )(parts)
    return logits, loss[0, 0]


# speedup vs baseline: 1.0214x; 1.0095x over previous
import functools

import jax
import jax.numpy as jnp
from jax.experimental import pallas as pl
from jax.experimental.pallas import tpu as pltpu

_ROWS = 4096


def _table_kernel(tok_ref, pos_ref, w_ref, b_ref, lse_ref, posb_ref):
    t = pl.program_id(0)
    posb = jax.lax.dot_general(
        pos_ref[pl.ds(t, 1), :], w_ref[...], (((1,), (0,)), ((), ())),
        preferred_element_type=jnp.float32,
        precision=jax.lax.Precision.DEFAULT) + b_ref[...]
    slab = jax.lax.dot_general(
        tok_ref[...], w_ref[...], (((1,), (0,)), ((), ())),
        preferred_element_type=jnp.float32,
        precision=jax.lax.Precision.DEFAULT) + posb
    m = jnp.max(slab, axis=1, keepdims=True)
    lse = jnp.log(jnp.sum(jnp.exp(slab - m), axis=1, keepdims=True)) + m
    lse_ref[...] = lse[None]
    posb_ref[...] = posb[None]


def _main_kernel(idx_ref, tgt_ref, a1_ref, a2_ref, pos_ref, mask_ref,
                 w_ref, logits_ref, part_ref):
    r, v = logits_ref.shape
    c = pos_ref.shape[1]

    ids = idx_ref[...]
    tgt = tgt_ref[...]
    vocab_iota = jax.lax.broadcasted_iota(jnp.int32, (r, v), 1)

    oh1 = (ids == vocab_iota).astype(jnp.bfloat16)
    g1 = jax.lax.dot_general(
        oh1, a1_ref[...], (((1,), (0,)), ((), ())),
        preferred_element_type=jnp.float32,
        precision=jax.lax.Precision.DEFAULT)
    x = g1[:, 0:c] + pos_ref[...]

    oh2 = (tgt == vocab_iota).astype(jnp.bfloat16)
    g2 = jax.lax.dot_general(
        oh2, a2_ref[...], (((1,), (0,)), ((), ())),
        preferred_element_type=jnp.float32,
        precision=jax.lax.Precision.DEFAULT)

    logits_ref[...] = jax.lax.dot_general(
        x, w_ref[...], (((1,), (0,)), ((), ())),
        preferred_element_type=jnp.float32,
        precision=jax.lax.Precision.DEFAULT)

    part = (jnp.sum((g1[:, c:c + 8] - g2[:, c:c + 8]) * mask_ref[...])
            - jnp.sum(x * g2[:, 0:c]))
    part_ref[...] = jnp.full((1, 1, 128), part, jnp.float32)


def _loss_reduce_kernel(part_ref, loss_ref, *, n):
    total = jnp.sum(part_ref[...][:, :, 0])
    loss_ref[...] = jnp.full((1, 1), total / n, jnp.float32)


def kernel(idx, targets, tok_table, pos_table, W, b):
    B, T = idx.shape
    V, C = tok_table.shape
    n = B * T
    r = _ROWS
    nblocks = n // r

    idx_r = idx.reshape(n, 1).astype(jnp.int32)
    tgt_r = targets.reshape(n, 1).astype(jnp.int32)
    b2 = b.reshape(1, V)

    lse_tab, posb = pl.pallas_call(
        _table_kernel,
        grid=(T,),
        in_specs=[
            pl.BlockSpec((V, C), lambda t: (0, 0)),
            pl.BlockSpec((T, C), lambda t: (0, 0)),
            pl.BlockSpec((C, V), lambda t: (0, 0)),
            pl.BlockSpec((1, V), lambda t: (0, 0)),
        ],
        out_specs=[
            pl.BlockSpec((1, V, 1), lambda t: (t, 0, 0)),
            pl.BlockSpec((1, 1, V), lambda t: (t, 0, 0)),
        ],
        out_shape=[
            jax.ShapeDtypeStruct((T, V, 1), jnp.float32),
            jax.ShapeDtypeStruct((T, 1, V), jnp.float32),
        ],
    )(tok_table, pos_table, W, b2)

    ones_col = jnp.ones((V, 1), jnp.float32)
    a1 = jnp.concatenate(
        [tok_table, ones_col, lse_tab.reshape(T, V).T],
        axis=1).astype(jnp.bfloat16)
    posw_t = posb.reshape(T, V) - b2
    a2 = jnp.concatenate(
        [W.T, b.reshape(V, 1), posw_t.T], axis=1).astype(jnp.bfloat16)
    w_aug = jnp.concatenate([W, b2], axis=0)
    pos_aug = jnp.concatenate(
        [pos_table, jnp.zeros((T, 1), jnp.float32)], axis=1)
    pos_tile = jnp.tile(pos_aug, (r // T, 1))
    mask_tile = jnp.tile(jnp.eye(T, dtype=jnp.float32), (r // T, 1))

    c1 = C + 1
    logits, parts = pl.pallas_call(
        _main_kernel,
        grid=(nblocks,),
        in_specs=[
            pl.BlockSpec((r, 1), lambda i: (i, 0)),
            pl.BlockSpec((r, 1), lambda i: (i, 0)),
            pl.BlockSpec((V, c1 + T), lambda i: (0, 0)),
            pl.BlockSpec((V, c1 + T), lambda i: (0, 0)),
            pl.BlockSpec((r, c1), lambda i: (0, 0)),
            pl.BlockSpec((r, T), lambda i: (0, 0)),
            pl.BlockSpec((c1, V), lambda i: (0, 0)),
        ],
        out_specs=[
            pl.BlockSpec((r, V), lambda i: (i, 0)),
            pl.BlockSpec((1, 1, 128), lambda i: (i, 0, 0)),
        ],
        out_shape=[
            jax.ShapeDtypeStruct((n, V), jnp.float32),
            jax.ShapeDtypeStruct((nblocks, 1, 128), jnp.float32),
        ],
        compiler_params=pltpu.CompilerParams(
            dimension_semantics=("parallel",)),
    )(idx_r, tgt_r, a1, a2, pos_tile, mask_tile, w_aug)

    loss = pl.pallas_call(
        functools.partial(_loss_reduce_kernel, n=n),
        out_shape=jax.ShapeDtypeStruct((1, 1), jnp.float32),
    )(parts)
    return logits, loss[0, 0]
